# baseline (speedup 1.0000x reference)
import functools

import jax
import jax.numpy as jnp
from jax import lax
from jax.experimental import pallas as pl
from jax.experimental.pallas import tpu as pltpu
from jax.experimental.pallas import tpu_sc as plsc

_NUM_CODES = 8192
_EMBED_DIM = 1024
_COMMIT = 0.25


_BM = 2048
_BN = 512


_WIN = 2736


def _bf16_rtne(x):
    u = lax.bitcast_convert_type(x, jnp.uint32)
    r = (u + jnp.uint32(0x7FFF) + ((u >> 16) & jnp.uint32(1))) \
        & jnp.uint32(0xFFFF0000)
    return lax.bitcast_convert_type(r, jnp.float32)


def _argmax_body(x_ref, w_ref, idx_ref, win_v, win_a, acc_v, acc_a):
    j = pl.program_id(1)
    nj = pl.num_programs(1)
    w = w_ref[...]
    nrm = jnp.sqrt(jnp.sum(w * w, axis=1, keepdims=True))
    cn = w / jnp.maximum(nrm, 1e-12)
    x = x_ref[...]
    xnrm = jnp.sqrt(jnp.sum(x * x, axis=1, keepdims=True))
    xn = x / jnp.maximum(xnrm, 1e-12)
    s = lax.dot_general(xn, cn, (((1,), (1,)), ((), ())),
                        preferred_element_type=jnp.float32,
                        precision=lax.Precision.DEFAULT)
    d = -s
    inf = jnp.float32(jnp.inf)
    big = jnp.int32(2**30)
    col0 = j * _BN
    cols = col0 + lax.broadcasted_iota(jnp.int32, d.shape, 1)
    wlo = col0 // _WIN
    whi = (col0 + _BN - 1) // _WIN
    in_lo = (cols // _WIN) == wlo
    d_lo = jnp.where(in_lo, d, inf)
    v_lo = jnp.min(d_lo, axis=1)
    a_lo = jnp.min(jnp.where(d_lo == v_lo[:, None], cols, big), axis=1)

    @pl.when(j == 0)
    def _():
        acc_v[...] = jnp.full((_BM,), inf, jnp.float32)
        acc_a[...] = jnp.zeros((_BM,), jnp.int32)
        win_v[...] = v_lo
        win_a[...] = a_lo

    @pl.when(j > 0)
    def _():
        take = v_lo < win_v[...]
        win_v[...] = jnp.where(take, v_lo, win_v[...])
        win_a[...] = jnp.where(take, a_lo, win_a[...])

    @pl.when(whi > wlo)
    def _():
        wv, wa = win_v[...], win_a[...]
        take = wv < acc_v[...]
        acc_v[...] = _bf16_rtne(jnp.where(take, wv, acc_v[...]))
        acc_a[...] = jnp.where(take, wa, acc_a[...])
        d_hi = jnp.where(in_lo, inf, d)
        v_hi = jnp.min(d_hi, axis=1)
        a_hi = jnp.min(jnp.where(d_hi == v_hi[:, None], cols, big), axis=1)
        win_v[...] = v_hi
        win_a[...] = a_hi

    @pl.when(j == nj - 1)
    def _():
        wv, wa = win_v[...], win_a[...]
        take = wv < acc_v[...]
        idx_ref[...] = jnp.where(take, wa, acc_a[...])


def _compute_indices(xf, W, interpret=False):
    m = xf.shape[0]
    grid = (m // _BM, _NUM_CODES // _BN)
    return pl.pallas_call(
        _argmax_body,
        grid=grid,
        in_specs=[
            pl.BlockSpec((_BM, _EMBED_DIM), lambda i, j: (i, 0)),
            pl.BlockSpec((_BN, _EMBED_DIM), lambda i, j: (j, 0)),
        ],
        out_specs=pl.BlockSpec((_BM,), lambda i, j: (i,)),
        out_shape=jax.ShapeDtypeStruct((m,), jnp.int32),
        scratch_shapes=[
            pltpu.VMEM((_BM,), jnp.float32),
            pltpu.VMEM((_BM,), jnp.int32),
            pltpu.VMEM((_BM,), jnp.float32),
            pltpu.VMEM((_BM,), jnp.int32),
        ],
        compiler_params=pltpu.CompilerParams(
            dimension_semantics=("parallel", "arbitrary")),
        interpret=interpret,
    )(xf, W)


_NC = 2
_NS = 16
_NW = _NC * _NS
_CHUNK = 64


def _sc_gather(W, idx):
    m = idx.shape[0]
    bpw = m // _NW
    nchunk = bpw // _CHUNK
    idx3 = idx.reshape(_NW, nchunk, _CHUNK)
    mesh = plsc.VectorSubcoreMesh(core_axis_name="c", subcore_axis_name="s")

    @functools.partial(
        pl.kernel,
        mesh=mesh,
        out_type=jax.ShapeDtypeStruct((m, _EMBED_DIM), jnp.float32),
        scratch_types=[
            pltpu.VMEM((nchunk, _CHUNK), jnp.int32),
            pltpu.VMEM((_CHUNK, _EMBED_DIM), jnp.float32),
            pltpu.SemaphoreType.DMA,
        ],
    )
    def k(w_hbm, idx_hbm, out_hbm, idx_v, rows_v, sem):
        wid = lax.axis_index("s") * _NC + lax.axis_index("c")
        base = wid * bpw
        pltpu.sync_copy(idx_hbm.at[wid], idx_v)
        for c in range(nchunk):
            pltpu.async_copy(w_hbm.at[idx_v.at[c]], rows_v, sem).wait()
            pltpu.sync_copy(rows_v, out_hbm.at[pl.ds(base + c * _CHUNK, _CHUNK)])

    return k(W, idx3)


_BR = 512


def _finalize_body(idx_ref, x_ref, q_ref, qst_ref, loss_ref, perp_ref,
                   acc, cnt):
    i = pl.program_id(0)
    ni = pl.num_programs(0)
    x = x_ref[...]
    q = _bf16_rtne(q_ref[...])
    d = q - x
    qst_ref[...] = x + d
    part = jnp.sum(d * d)
    idx_blk = idx_ref[0, 0, :]
    hist = jnp.sum(
        (idx_blk[:, None]
         == lax.broadcasted_iota(jnp.int32, (_BR, _NUM_CODES), 1)
         ).astype(jnp.float32),
        axis=0)

    @pl.when(i == 0)
    def _():
        acc[0] = part
        cnt[...] = hist

    @pl.when(i > 0)
    def _():
        acc[0] = acc[0] + part
        cnt[...] = cnt[...] + hist

    @pl.when(i == ni - 1)
    def _():
        n_rows = ni * _BR
        mse = acc[0] / (n_rows * _EMBED_DIM)
        loss_ref[...] = jnp.reshape(mse + _COMMIT * mse, (1, 1))
        p = cnt[...] / n_rows
        perp_ref[...] = jnp.reshape(
            jnp.exp(-jnp.sum(p * jnp.log(p + 1e-10))), (1, 1))


def _finalize(idx, xf, q, interpret=False):
    m = xf.shape[0]
    ni = m // _BR
    idx3 = idx.reshape(ni, 1, _BR)
    return pl.pallas_call(
        _finalize_body,
        grid=(ni,),
        in_specs=[
            pl.BlockSpec((1, 1, _BR), lambda i: (i, 0, 0)),
            pl.BlockSpec((_BR, _EMBED_DIM), lambda i: (i, 0)),
            pl.BlockSpec((_BR, _EMBED_DIM), lambda i: (i, 0)),
        ],
        out_specs=[
            pl.BlockSpec((_BR, _EMBED_DIM), lambda i: (i, 0)),
            pl.BlockSpec((1, 1), lambda i: (0, 0)),
            pl.BlockSpec((1, 1), lambda i: (0, 0)),
        ],
        out_shape=[
            jax.ShapeDtypeStruct((m, _EMBED_DIM), jnp.float32),
            jax.ShapeDtypeStruct((1, 1), jnp.float32),
            jax.ShapeDtypeStruct((1, 1), jnp.float32),
        ],
        scratch_shapes=[
            pltpu.SMEM((1,), jnp.float32),
            pltpu.VMEM((_NUM_CODES,), jnp.float32),
        ],
        compiler_params=pltpu.CompilerParams(
            dimension_semantics=("arbitrary",)),
        interpret=interpret,
    )(idx3, xf, q)


def kernel(inputs, W):
    b, n, d = inputs.shape
    xf = inputs.reshape(-1, d)
    idx = _compute_indices(xf, W)
    q = _sc_gather(W, idx)
    qst, loss, perp = _finalize(idx, xf, q)
    return qst.reshape(b, n, d), loss[0, 0], perp[0, 0]


# speedup vs baseline: 1.3658x; 1.3658x over previous
import functools

import jax
import jax.numpy as jnp
from jax import lax
from jax.experimental import pallas as pl
from jax.experimental.pallas import tpu as pltpu
from jax.experimental.pallas import tpu_sc as plsc

_NUM_CODES = 8192
_EMBED_DIM = 1024
_COMMIT = 0.25


_BM = 2048
_BN = 512


_WIN = 2736


def _bf16_rtne(x):
    u = lax.bitcast_convert_type(x, jnp.uint32)
    r = (u + jnp.uint32(0x7FFF) + ((u >> 16) & jnp.uint32(1))) \
        & jnp.uint32(0xFFFF0000)
    return lax.bitcast_convert_type(r, jnp.float32)


def _norm_w_body(w_ref, cn_ref):
    w = w_ref[...]
    nrm = jnp.sqrt(jnp.sum(w * w, axis=1, keepdims=True))
    cn_ref[...] = (w / jnp.maximum(nrm, 1e-12)).astype(jnp.bfloat16)


def _normalize_codebook(W, interpret=False):
    return pl.pallas_call(
        _norm_w_body,
        grid=(_NUM_CODES // _BN,),
        in_specs=[pl.BlockSpec((_BN, _EMBED_DIM), lambda j: (j, 0))],
        out_specs=pl.BlockSpec((_BN, _EMBED_DIM), lambda j: (j, 0)),
        out_shape=jax.ShapeDtypeStruct((_NUM_CODES, _EMBED_DIM),
                                       jnp.bfloat16),
        interpret=interpret,
    )(W)


_B1 = _WIN // _BN
_B2 = (2 * _WIN) // _BN


def _argmax_body(x_ref, cn_ref, idx_ref, xn_s, accv, accb, gv, ga):
    j = pl.program_id(1)
    nj = pl.num_programs(1)
    ninf = jnp.float32(-jnp.inf)
    big = jnp.int32(2**30)

    @pl.when(j == 0)
    def _():
        x = x_ref[...]
        xnrm = jnp.sqrt(jnp.sum(x * x, axis=1, keepdims=True))
        xn_s[...] = (x / jnp.maximum(xnrm, 1e-12)).astype(jnp.bfloat16)
        gv[...] = jnp.full((_BM,), ninf, jnp.float32)
        ga[...] = jnp.zeros((_BM,), jnp.int32)

    s = lax.dot_general(xn_s[...], cn_ref[...], (((1,), (1,)), ((), ())),
                        preferred_element_type=jnp.float32)
    lane = lax.broadcasted_iota(jnp.int32, s.shape, 1)

    def collapse_and_merge(av, ab):
        v = jnp.max(av, axis=1)
        code = ab * _BN + lane
        a = jnp.min(jnp.where(av == v[:, None], code, big), axis=1)
        take = v > gv[...]
        gv[...] = _bf16_rtne(jnp.where(take, v, gv[...]))
        ga[...] = jnp.where(take, a, ga[...])

    def boundary(bnd):
        in_lo = (j * _BN + lane) < bnd
        s_lo = jnp.where(in_lo, s, ninf)
        take = s_lo > accv[...]
        av = jnp.where(take, s_lo, accv[...])
        ab = jnp.where(take, j, accb[...])
        collapse_and_merge(av, ab)
        accv[...] = jnp.where(in_lo, ninf, s)
        accb[...] = jnp.full(s.shape, j, jnp.int32)

    @pl.when(j == 0)
    def _():
        accv[...] = s
        accb[...] = jnp.zeros(s.shape, jnp.int32)

    @pl.when((j > 0) & (j != _B1) & (j != _B2))
    def _():
        take = s > accv[...]
        accv[...] = jnp.where(take, s, accv[...])
        accb[...] = jnp.where(take, j, accb[...])

    @pl.when(j == _B1)
    def _():
        boundary(_WIN)

    @pl.when(j == _B2)
    def _():
        boundary(2 * _WIN)

    @pl.when(j == nj - 1)
    def _():
        collapse_and_merge(accv[...], accb[...])
        idx_ref[...] = ga[...]


def _compute_indices(xf, cn, interpret=False):
    m = xf.shape[0]
    grid = (m // _BM, _NUM_CODES // _BN)
    return pl.pallas_call(
        _argmax_body,
        grid=grid,
        in_specs=[
            pl.BlockSpec((_BM, _EMBED_DIM), lambda i, j: (i, 0)),
            pl.BlockSpec((_BN, _EMBED_DIM), lambda i, j: (j, 0)),
        ],
        out_specs=pl.BlockSpec((_BM,), lambda i, j: (i,)),
        out_shape=jax.ShapeDtypeStruct((m,), jnp.int32),
        scratch_shapes=[
            pltpu.VMEM((_BM, _EMBED_DIM), jnp.bfloat16),
            pltpu.VMEM((_BM, _BN), jnp.float32),
            pltpu.VMEM((_BM, _BN), jnp.int32),
            pltpu.VMEM((_BM,), jnp.float32),
            pltpu.VMEM((_BM,), jnp.int32),
        ],
        compiler_params=pltpu.CompilerParams(
            dimension_semantics=("parallel", "arbitrary")),
        interpret=interpret,
    )(xf, cn)


_NC = 2
_NS = 16
_NW = _NC * _NS
_CHUNK = 64


def _sc_gather(W, idx):
    m = idx.shape[0]
    bpw = m // _NW
    nchunk = bpw // _CHUNK
    idx3 = idx.reshape(_NW, nchunk, _CHUNK)
    mesh = plsc.VectorSubcoreMesh(core_axis_name="c", subcore_axis_name="s")

    @functools.partial(
        pl.kernel,
        mesh=mesh,
        out_type=jax.ShapeDtypeStruct((m, _EMBED_DIM), jnp.float32),
        scratch_types=[
            pltpu.VMEM((nchunk, _CHUNK), jnp.int32),
            pltpu.VMEM((_CHUNK, _EMBED_DIM), jnp.float32),
            pltpu.SemaphoreType.DMA,
        ],
    )
    def k(w_hbm, idx_hbm, out_hbm, idx_v, rows_v, sem):
        wid = lax.axis_index("s") * _NC + lax.axis_index("c")
        base = wid * bpw
        pltpu.sync_copy(idx_hbm.at[wid], idx_v)
        for c in range(nchunk):
            pltpu.async_copy(w_hbm.at[idx_v.at[c]], rows_v, sem).wait()
            pltpu.sync_copy(rows_v, out_hbm.at[pl.ds(base + c * _CHUNK, _CHUNK)])

    return k(W, idx3)


_BR = 512


def _finalize_body(idx_ref, x_ref, q_ref, qst_ref, loss_ref, perp_ref,
                   acc, cnt):
    i = pl.program_id(0)
    ni = pl.num_programs(0)
    x = x_ref[...]
    q = _bf16_rtne(q_ref[...])
    d = q - x
    qst_ref[...] = x + d
    part = jnp.sum(d * d)
    idx_blk = idx_ref[0, 0, :]
    hist = jnp.sum(
        (idx_blk[:, None]
         == lax.broadcasted_iota(jnp.int32, (_BR, _NUM_CODES), 1)
         ).astype(jnp.float32),
        axis=0)

    @pl.when(i == 0)
    def _():
        acc[0] = part
        cnt[...] = hist

    @pl.when(i > 0)
    def _():
        acc[0] = acc[0] + part
        cnt[...] = cnt[...] + hist

    @pl.when(i == ni - 1)
    def _():
        n_rows = ni * _BR
        mse = acc[0] / (n_rows * _EMBED_DIM)
        loss_ref[...] = jnp.reshape(mse + _COMMIT * mse, (1, 1))
        p = cnt[...] / n_rows
        perp_ref[...] = jnp.reshape(
            jnp.exp(-jnp.sum(p * jnp.log(p + 1e-10))), (1, 1))


def _finalize(idx, xf, q, interpret=False):
    m = xf.shape[0]
    ni = m // _BR
    idx3 = idx.reshape(ni, 1, _BR)
    return pl.pallas_call(
        _finalize_body,
        grid=(ni,),
        in_specs=[
            pl.BlockSpec((1, 1, _BR), lambda i: (i, 0, 0)),
            pl.BlockSpec((_BR, _EMBED_DIM), lambda i: (i, 0)),
            pl.BlockSpec((_BR, _EMBED_DIM), lambda i: (i, 0)),
        ],
        out_specs=[
            pl.BlockSpec((_BR, _EMBED_DIM), lambda i: (i, 0)),
            pl.BlockSpec((1, 1), lambda i: (0, 0)),
            pl.BlockSpec((1, 1), lambda i: (0, 0)),
        ],
        out_shape=[
            jax.ShapeDtypeStruct((m, _EMBED_DIM), jnp.float32),
            jax.ShapeDtypeStruct((1, 1), jnp.float32),
            jax.ShapeDtypeStruct((1, 1), jnp.float32),
        ],
        scratch_shapes=[
            pltpu.SMEM((1,), jnp.float32),
            pltpu.VMEM((_NUM_CODES,), jnp.float32),
        ],
        compiler_params=pltpu.CompilerParams(
            dimension_semantics=("arbitrary",)),
        interpret=interpret,
    )(idx3, xf, q)


def kernel(inputs, W):
    b, n, d = inputs.shape
    xf = inputs.reshape(-1, d)
    cn = _normalize_codebook(W)
    idx = _compute_indices(xf, cn)
    q = _sc_gather(W, idx)
    qst, loss, perp = _finalize(idx, xf, q)
    return qst.reshape(b, n, d), loss[0, 0], perp[0, 0]
